# Initial kernel scaffold; baseline (speedup 1.0000x reference)
import jax
import jax.numpy as jnp
from jax.experimental import pallas as pl


def kernel(x, y, senders, receivers, rel_pos, a, W1, b1, W2, b2, bn1_scale, bn1_offset, bn1_mean, bn1_var, bn2_scale, bn2_offset, bn2_mean, bn2_var):
    raise NotImplementedError("write your pallas kernel here")


# speedup vs baseline: 2.1203x; 2.1203x over previous
import functools
import jax
import jax.numpy as jnp
from jax import lax
from jax.experimental import pallas as pl
from jax.experimental.pallas import tpu as pltpu
from jax.experimental.pallas import tpu_sc as plsc

N = 50000
E = 800000
CH = 32
INTER = 64
KC = 16

NC = 2
NS = 16
NW = NC * NS

GW = 1000
NWIN = E // GW
GWS = 200
NWINS = E // GWS
NPT = N // NS

BE = 2000
BN = 2000


def _make_sc_gather(D):
  mesh = plsc.VectorSubcoreMesh(core_axis_name="c", subcore_axis_name="s")
  wpt = NWIN // NW

  @functools.partial(
      pl.kernel, mesh=mesh,
      out_type=jax.ShapeDtypeStruct((E, D), jnp.float32),
      scratch_types=[
          pltpu.VMEM((GW,), jnp.int32),
          pltpu.VMEM((GW, D), jnp.float32),
          pltpu.SemaphoreType.DMA,
      ],
      compiler_params=pltpu.CompilerParams(use_tc_tiling_on_sc=False),
  )
  def k(table_hbm, idx_hbm, out_hbm, idx_v, rows_v, sem):
    wid = lax.axis_index("s") * NC + lax.axis_index("c")

    def step(t, carry):
      base = (wid + NW * t) * GW
      pltpu.sync_copy(idx_hbm.at[pl.ds(base, GW)], idx_v)
      pltpu.async_copy(table_hbm.at[idx_v], rows_v, sem).wait()
      pltpu.sync_copy(rows_v, out_hbm.at[pl.ds(base, GW)])
      return carry

    lax.fori_loop(0, wpt, step, 0)

  return k


def _make_sc_scatter(D):
  mesh = plsc.VectorSubcoreMesh(core_axis_name="c", subcore_axis_name="s")
  wpt = NWINS // NS

  @functools.partial(
      pl.kernel, mesh=mesh,
      out_type=jax.ShapeDtypeStruct((NC, N, D), jnp.float32),
      scratch_types=[
          pltpu.VMEM((GWS,), jnp.int32),
          pltpu.VMEM((GWS, D), jnp.float32),
          pltpu.VMEM_SHARED((N, D), jnp.float32),
      ],
      compiler_params=pltpu.CompilerParams(use_tc_tiling_on_sc=False),
  )
  def k(recv_hbm, msg_hbm, zero_hbm, sum_hbm, idx_v, upd_v, acc):
    c = lax.axis_index("c")
    s = lax.axis_index("s")
    pltpu.sync_copy(zero_hbm.at[pl.ds(s * NPT, NPT)],
                    acc.at[pl.ds(s * NPT, NPT)])
    plsc.subcore_barrier()

    def step(t, carry):
      base = (s + NS * t) * GWS
      pltpu.sync_copy(recv_hbm.at[pl.ds(base, GWS)], idx_v)
      pltpu.sync_copy(msg_hbm.at[c, pl.ds(base, GWS)], upd_v)
      pltpu.sync_copy(upd_v, acc.at[idx_v], add=True)
      return carry

    lax.fori_loop(0, wpt, step, 0)
    plsc.subcore_barrier()
    pltpu.sync_copy(acc.at[pl.ds(s * NPT, NPT)],
                    sum_hbm.at[c, pl.ds(s * NPT, NPT)])

  return k


def _msg_body(w_ref, rx_ref, ry_ref, a_ref, *rest):
  feat_refs, out_ref = rest[:-1], rest[-1]
  feat = jnp.concatenate([f[...] for f in feat_refs], axis=1)
  outdim = w_ref.shape[1] // KC
  gx = jnp.clip(jnp.floor(rx_ref[...] * 4.0), 0.0, 3.0)
  gy = jnp.clip(jnp.floor(ry_ref[...] * 4.0), 0.0, 3.0)
  cf = gx * 4.0 + gy
  p = jnp.dot(feat, w_ref[...], preferred_element_type=jnp.float32)
  msg = jnp.zeros((feat.shape[0], outdim), jnp.float32)
  for kk in range(KC):
    msg = msg + jnp.where(cf == float(kk),
                          p[:, kk * outdim:(kk + 1) * outdim], 0.0)
  msg = msg * a_ref[...]
  half = outdim // 2
  out_ref[0] = msg[:, :half]
  out_ref[1] = msg[:, half:]


def _make_tc_msg(indim, outdim, nfeat):
  nb = E // BE
  half = outdim // 2
  feat_specs = [
      pl.BlockSpec((BE, indim // nfeat), lambda i: (i, 0))
      for _ in range(nfeat)
  ]
  return pl.pallas_call(
      _msg_body,
      grid=(nb,),
      in_specs=[
          pl.BlockSpec((indim, KC * outdim), lambda i: (0, 0)),
          pl.BlockSpec((BE, 1), lambda i: (i, 0)),
          pl.BlockSpec((BE, 1), lambda i: (i, 0)),
          pl.BlockSpec((BE, 1), lambda i: (i, 0)),
      ] + feat_specs,
      out_specs=pl.BlockSpec((NC, BE, half), lambda i: (0, i, 0)),
      out_shape=jax.ShapeDtypeStruct((NC, E, half), jnp.float32),
  )


def _bnrelu_body(sum_ref, s_ref, o_ref, outa_ref, outb_ref):
  h = jnp.concatenate([sum_ref[0], sum_ref[1]], axis=1)
  xl = jnp.maximum(h * s_ref[...] + o_ref[...], 0.0)
  outa_ref[...] = xl[:, :CH]
  outb_ref[...] = xl[:, CH:]


_tc_bnrelu = pl.pallas_call(
    _bnrelu_body,
    grid=(N // BN,),
    in_specs=[
        pl.BlockSpec((NC, BN, CH), lambda i: (0, i, 0)),
        pl.BlockSpec((1, INTER), lambda i: (0, 0)),
        pl.BlockSpec((1, INTER), lambda i: (0, 0)),
    ],
    out_specs=[
        pl.BlockSpec((BN, CH), lambda i: (i, 0)),
        pl.BlockSpec((BN, CH), lambda i: (i, 0)),
    ],
    out_shape=[
        jax.ShapeDtypeStruct((N, CH), jnp.float32),
        jax.ShapeDtypeStruct((N, CH), jnp.float32),
    ],
)


def _blend_body(x_ref, y_ref, sum_ref, s_ref, o_ref, out_ref):
  h = jnp.concatenate([sum_ref[0], sum_ref[1]], axis=1)
  z = h * s_ref[...] + o_ref[...]
  wei = jax.nn.sigmoid(z)
  out_ref[...] = 2.0 * x_ref[...] * wei + 2.0 * y_ref[...] * (1.0 - wei)


_tc_blend = pl.pallas_call(
    _blend_body,
    grid=(N // BN,),
    in_specs=[
        pl.BlockSpec((BN, CH), lambda i: (i, 0)),
        pl.BlockSpec((BN, CH), lambda i: (i, 0)),
        pl.BlockSpec((NC, BN, CH // 2), lambda i: (0, i, 0)),
        pl.BlockSpec((1, CH), lambda i: (0, 0)),
        pl.BlockSpec((1, CH), lambda i: (0, 0)),
    ],
    out_specs=pl.BlockSpec((BN, CH), lambda i: (i, 0)),
    out_shape=jax.ShapeDtypeStruct((N, CH), jnp.float32),
)

_sc_gather64 = _make_sc_gather(2 * CH)
_sc_gather32 = _make_sc_gather(CH)
_sc_scatter32 = _make_sc_scatter(CH)
_sc_scatter16 = _make_sc_scatter(CH // 2)
_tc_msg1 = _make_tc_msg(2 * CH, INTER, 1)
_tc_msg2 = _make_tc_msg(INTER, CH, 2)


def kernel(x, y, senders, receivers, rel_pos, a, W1, b1, W2, b2,
           bn1_scale, bn1_offset, bn1_mean, bn1_var,
           bn2_scale, bn2_offset, bn2_mean, bn2_var):
  eps = 1e-5
  s1 = bn1_scale / jnp.sqrt(bn1_var + eps)
  o1 = (b1 - bn1_mean) * s1 + bn1_offset
  s2 = bn2_scale / jnp.sqrt(bn2_var + eps)
  o2 = (b2 - bn2_mean) * s2 + bn2_offset

  xa = jnp.concatenate([x, y], axis=1)
  rx3 = rel_pos[:, 0:1]
  ry3 = rel_pos[:, 1:2]
  a3 = a.reshape(E, 1)
  w1c = W1.transpose(1, 0, 2).reshape(2 * CH, KC * INTER)
  w2c = W2.transpose(1, 0, 2).reshape(INTER, KC * CH)
  zeros32 = jnp.zeros((N, CH), jnp.float32)
  zeros16 = jnp.zeros((N, CH // 2), jnp.float32)

  feat1 = _sc_gather64(xa, senders)
  msg1 = _tc_msg1(w1c, rx3, ry3, a3, feat1)
  sum1 = _sc_scatter32(receivers, msg1, zeros32)
  xla_, xlb = _tc_bnrelu(sum1, s1.reshape(1, INTER), o1.reshape(1, INTER))
  feat2a = _sc_gather32(xla_, senders)
  feat2b = _sc_gather32(xlb, senders)
  msg2 = _tc_msg2(w2c, rx3, ry3, a3, feat2a, feat2b)
  sum2 = _sc_scatter16(receivers, msg2, zeros16)
  return _tc_blend(x, y, sum2, s2.reshape(1, CH), o2.reshape(1, CH))


# speedup vs baseline: 2.1987x; 1.0370x over previous
import functools
import jax
import jax.numpy as jnp
from jax import lax
from jax.experimental import pallas as pl
from jax.experimental.pallas import tpu as pltpu
from jax.experimental.pallas import tpu_sc as plsc

N = 50000
E = 800000
CH = 32
INTER = 64
KC = 16

NC = 2
NS = 16
NW = NC * NS

GW = 1000
NWIN = E // GW
GWS = 200
NWINS = E // GWS
NPT = N // NS

BE = 2000
BN = 2000


def _make_sc_gather(D):
  mesh = plsc.VectorSubcoreMesh(core_axis_name="c", subcore_axis_name="s")
  wpt = NWIN // NW

  @functools.partial(
      pl.kernel, mesh=mesh,
      out_type=jax.ShapeDtypeStruct((E, D), jnp.float32),
      scratch_types=[
          pltpu.VMEM((GW,), jnp.int32),
          pltpu.VMEM((GW, D), jnp.float32),
          pltpu.SemaphoreType.DMA,
      ],
      compiler_params=pltpu.CompilerParams(use_tc_tiling_on_sc=False),
  )
  def k(table_hbm, idx_hbm, out_hbm, idx_v, rows_v, sem):
    wid = lax.axis_index("s") * NC + lax.axis_index("c")

    def step(t, carry):
      base = (wid + NW * t) * GW
      pltpu.sync_copy(idx_hbm.at[pl.ds(base, GW)], idx_v)
      pltpu.async_copy(table_hbm.at[idx_v], rows_v, sem).wait()
      pltpu.sync_copy(rows_v, out_hbm.at[pl.ds(base, GW)])
      return carry

    lax.fori_loop(0, wpt, step, 0)

  return k


def _make_sc_scatter(D):
  mesh = plsc.VectorSubcoreMesh(core_axis_name="c", subcore_axis_name="s")
  wpt = NWINS // NS

  @functools.partial(
      pl.kernel, mesh=mesh,
      out_type=jax.ShapeDtypeStruct((NC, N, D), jnp.float32),
      scratch_types=[
          pltpu.VMEM((GWS,), jnp.int32),
          pltpu.VMEM((GWS, D), jnp.float32),
          pltpu.VMEM_SHARED((N, D), jnp.float32),
      ],
      compiler_params=pltpu.CompilerParams(use_tc_tiling_on_sc=False),
  )
  def k(recv_hbm, msg_hbm, zero_hbm, sum_hbm, idx_v, upd_v, acc):
    c = lax.axis_index("c")
    s = lax.axis_index("s")
    pltpu.sync_copy(zero_hbm.at[pl.ds(s * NPT, NPT)],
                    acc.at[pl.ds(s * NPT, NPT)])
    plsc.subcore_barrier()

    def step(t, carry):
      base = (s + NS * t) * GWS
      pltpu.sync_copy(recv_hbm.at[pl.ds(base, GWS)], idx_v)
      pltpu.sync_copy(msg_hbm.at[c, pl.ds(base, GWS)], upd_v)
      pltpu.sync_copy(upd_v, acc.at[idx_v], add=True)
      return carry

    lax.fori_loop(0, wpt, step, 0)
    plsc.subcore_barrier()
    pltpu.sync_copy(acc.at[pl.ds(s * NPT, NPT)],
                    sum_hbm.at[c, pl.ds(s * NPT, NPT)])

  return k


def _msg_body(w_ref, rx_ref, ry_ref, a_ref, *rest):
  feat_refs, out_ref = rest[:-1], rest[-1]
  feat = jnp.concatenate(
      [f[...] for f in feat_refs], axis=1).astype(jnp.bfloat16)
  outdim = w_ref.shape[1]
  gx = jnp.clip(jnp.floor(rx_ref[...] * 4.0), 0.0, 3.0)
  gy = jnp.clip(jnp.floor(ry_ref[...] * 4.0), 0.0, 3.0)
  cf = gx * 4.0 + gy
  zero = jnp.zeros_like(feat)
  z = jnp.concatenate(
      [jnp.where(cf == float(kk), feat, zero) for kk in range(KC)], axis=1)
  msg = jnp.dot(z, w_ref[...], preferred_element_type=jnp.float32)
  msg = msg * a_ref[...]
  half = outdim // 2
  out_ref[0] = msg[:, :half]
  out_ref[1] = msg[:, half:]


def _make_tc_msg(indim, outdim, nfeat):
  nb = E // BE
  half = outdim // 2
  feat_specs = [
      pl.BlockSpec((BE, indim // nfeat), lambda i: (i, 0))
      for _ in range(nfeat)
  ]
  return pl.pallas_call(
      _msg_body,
      grid=(nb,),
      in_specs=[
          pl.BlockSpec((KC * indim, outdim), lambda i: (0, 0)),
          pl.BlockSpec((BE, 1), lambda i: (i, 0)),
          pl.BlockSpec((BE, 1), lambda i: (i, 0)),
          pl.BlockSpec((BE, 1), lambda i: (i, 0)),
      ] + feat_specs,
      out_specs=pl.BlockSpec((NC, BE, half), lambda i: (0, i, 0)),
      out_shape=jax.ShapeDtypeStruct((NC, E, half), jnp.float32),
  )


def _bnrelu_body(sum_ref, s_ref, o_ref, outa_ref, outb_ref):
  h = jnp.concatenate([sum_ref[0], sum_ref[1]], axis=1)
  xl = jnp.maximum(h * s_ref[...] + o_ref[...], 0.0)
  outa_ref[...] = xl[:, :CH]
  outb_ref[...] = xl[:, CH:]


_tc_bnrelu = pl.pallas_call(
    _bnrelu_body,
    grid=(N // BN,),
    in_specs=[
        pl.BlockSpec((NC, BN, CH), lambda i: (0, i, 0)),
        pl.BlockSpec((1, INTER), lambda i: (0, 0)),
        pl.BlockSpec((1, INTER), lambda i: (0, 0)),
    ],
    out_specs=[
        pl.BlockSpec((BN, CH), lambda i: (i, 0)),
        pl.BlockSpec((BN, CH), lambda i: (i, 0)),
    ],
    out_shape=[
        jax.ShapeDtypeStruct((N, CH), jnp.float32),
        jax.ShapeDtypeStruct((N, CH), jnp.float32),
    ],
)


def _blend_body(x_ref, y_ref, sum_ref, s_ref, o_ref, out_ref):
  h = jnp.concatenate([sum_ref[0], sum_ref[1]], axis=1)
  z = h * s_ref[...] + o_ref[...]
  wei = jax.nn.sigmoid(z)
  out_ref[...] = 2.0 * x_ref[...] * wei + 2.0 * y_ref[...] * (1.0 - wei)


_tc_blend = pl.pallas_call(
    _blend_body,
    grid=(N // BN,),
    in_specs=[
        pl.BlockSpec((BN, CH), lambda i: (i, 0)),
        pl.BlockSpec((BN, CH), lambda i: (i, 0)),
        pl.BlockSpec((NC, BN, CH // 2), lambda i: (0, i, 0)),
        pl.BlockSpec((1, CH), lambda i: (0, 0)),
        pl.BlockSpec((1, CH), lambda i: (0, 0)),
    ],
    out_specs=pl.BlockSpec((BN, CH), lambda i: (i, 0)),
    out_shape=jax.ShapeDtypeStruct((N, CH), jnp.float32),
)

_sc_gather64 = _make_sc_gather(2 * CH)
_sc_gather32 = _make_sc_gather(CH)
_sc_scatter32 = _make_sc_scatter(CH)
_sc_scatter16 = _make_sc_scatter(CH // 2)
_tc_msg1 = _make_tc_msg(2 * CH, INTER, 1)
_tc_msg2 = _make_tc_msg(INTER, CH, 2)


def kernel(x, y, senders, receivers, rel_pos, a, W1, b1, W2, b2,
           bn1_scale, bn1_offset, bn1_mean, bn1_var,
           bn2_scale, bn2_offset, bn2_mean, bn2_var):
  eps = 1e-5
  s1 = bn1_scale / jnp.sqrt(bn1_var + eps)
  o1 = (b1 - bn1_mean) * s1 + bn1_offset
  s2 = bn2_scale / jnp.sqrt(bn2_var + eps)
  o2 = (b2 - bn2_mean) * s2 + bn2_offset

  xa = jnp.concatenate([x, y], axis=1)
  rx3 = rel_pos[:, 0:1]
  ry3 = rel_pos[:, 1:2]
  a3 = a.reshape(E, 1)
  w1c = W1.reshape(KC * 2 * CH, INTER).astype(jnp.bfloat16)
  w2c = W2.reshape(KC * INTER, CH).astype(jnp.bfloat16)
  zeros32 = jnp.zeros((N, CH), jnp.float32)
  zeros16 = jnp.zeros((N, CH // 2), jnp.float32)

  feat1 = _sc_gather64(xa, senders)
  msg1 = _tc_msg1(w1c, rx3, ry3, a3, feat1)
  sum1 = _sc_scatter32(receivers, msg1, zeros32)
  xla_, xlb = _tc_bnrelu(sum1, s1.reshape(1, INTER), o1.reshape(1, INTER))
  feat2a = _sc_gather32(xla_, senders)
  feat2b = _sc_gather32(xlb, senders)
  msg2 = _tc_msg2(w2c, rx3, ry3, a3, feat2a, feat2b)
  sum2 = _sc_scatter16(receivers, msg2, zeros16)
  return _tc_blend(x, y, sum2, s2.reshape(1, CH), o2.reshape(1, CH))
